# baseline (speedup 1.0000x reference)
import functools

import jax
import jax.numpy as jnp
from jax import lax
from jax.experimental import pallas as pl
from jax.experimental.pallas import tpu as pltpu
from jax.experimental.pallas import tpu_sc as plsc

D = 64
R = 8

_info = plsc.get_sparse_core_info()
_NC = _info.num_cores
_NS = _info.num_subcores
_NW = _NC * _NS


@functools.lru_cache(maxsize=None)
def _build(b, s):
    rows_w = b // _NW
    n_slabs = rows_w // R
    assert n_slabs % 2 == 0 and n_slabs >= 2
    mesh = plsc.VectorSubcoreMesh(core_axis_name="c", subcore_axis_name="s")

    @functools.partial(
        pl.kernel,
        mesh=mesh,
        compiler_params=pltpu.CompilerParams(use_tc_tiling_on_sc=False),
        out_type=jax.ShapeDtypeStruct((b, s, D), jnp.float32),
        scratch_types=[
            pltpu.VMEM((rows_w, s), jnp.int32),
            pltpu.VMEM((R, s, D), jnp.float32),
            pltpu.VMEM((R, s, D), jnp.float32),
            pltpu.SemaphoreType.DMA,
            pltpu.SemaphoreType.DMA,
            pltpu.SemaphoreType.DMA,
            pltpu.SemaphoreType.DMA,
        ],
    )
    def emb(idx_hbm, table_hbm, out_hbm, idx_v, buf_a, buf_b,
            gsem_a, gsem_b, wsem_a, wsem_b):
        wid = lax.axis_index("s") * _NC + lax.axis_index("c")
        base = wid * rows_w
        pltpu.sync_copy(idx_hbm.at[pl.ds(base, rows_w)], idx_v)

        def fire_gather(slab, buf, sem):
            for t in range(R):
                pltpu.async_copy(
                    table_hbm.at[idx_v.at[slab * R + t]], buf.at[t], sem)

        def wait_gather(buf, sem):
            pltpu.make_async_copy(out_hbm.at[pl.ds(0, R)], buf, sem).wait()

        def fire_write(slab, buf, sem):
            return pltpu.async_copy(
                buf, out_hbm.at[pl.ds(base + slab * R, R)], sem)

        def wait_write(buf, sem):
            pltpu.make_async_copy(
                buf, out_hbm.at[pl.ds(base, R)], sem).wait()

        fire_gather(0, buf_a, gsem_a)

        @pl.loop(0, n_slabs, step=2)
        def body(ja):
            jb = ja + 1
            wait_gather(buf_a, gsem_a)

            @pl.when(ja > 0)
            def _():
                wait_write(buf_b, wsem_b)

            fire_gather(jb, buf_b, gsem_b)
            hw_a = fire_write(ja, buf_a, wsem_a)
            wait_gather(buf_b, gsem_b)
            hw_a.wait()

            @pl.when(jb + 1 < n_slabs)
            def _():
                fire_gather(ja + 2, buf_a, gsem_a)

            fire_write(jb, buf_b, wsem_b)

        wait_write(buf_b, wsem_b)

    return emb


def kernel(input_ids, table):
    b, s = input_ids.shape
    return _build(b, s)(input_ids, table)


# speedup vs baseline: 1.0512x; 1.0512x over previous
import functools

import jax
import jax.numpy as jnp
from jax import lax
from jax.experimental import pallas as pl
from jax.experimental.pallas import tpu as pltpu
from jax.experimental.pallas import tpu_sc as plsc

D = 64
R = 8

_info = plsc.get_sparse_core_info()
_NC = _info.num_cores
_NS = _info.num_subcores
_NW = _NC * _NS


@functools.lru_cache(maxsize=None)
def _build(b, s):
    rows_w = b // _NW
    n_slabs = rows_w // R
    assert n_slabs % 2 == 0 and n_slabs >= 2
    mesh = plsc.VectorSubcoreMesh(core_axis_name="c", subcore_axis_name="s")

    @functools.partial(
        pl.kernel,
        mesh=mesh,
        compiler_params=pltpu.CompilerParams(use_tc_tiling_on_sc=False),
        out_type=jax.ShapeDtypeStruct((b, s, D), jnp.float32),
        scratch_types=[
            pltpu.VMEM((rows_w, s), jnp.int32),
            pltpu.VMEM((R, s, D), jnp.float32),
            pltpu.VMEM((R, s, D), jnp.float32),
            pltpu.SemaphoreType.DMA,
            pltpu.SemaphoreType.DMA,
            pltpu.SemaphoreType.DMA,
            pltpu.SemaphoreType.DMA,
        ],
    )
    def emb(idx_hbm, table_hbm, out_hbm, idx_v, buf_a, buf_b,
            gsem_a, gsem_b, wsem_a, wsem_b):
        wid = lax.axis_index("s") * _NC + lax.axis_index("c")
        base = wid * rows_w
        pltpu.sync_copy(idx_hbm.at[pl.ds(base, rows_w)], idx_v)

        def fire_gather(slab, buf, sem):
            for t in range(R):
                pltpu.async_copy(
                    table_hbm.at[idx_v.at[slab * R + t]], buf.at[t], sem)

        def wait_gather(buf, sem):
            pltpu.make_async_copy(out_hbm.at[pl.ds(0, R)], buf, sem).wait()

        def fire_write(slab, buf, sem):
            return pltpu.async_copy(
                buf, out_hbm.at[pl.ds(base + slab * R, R)], sem)

        def wait_write(buf, sem):
            pltpu.make_async_copy(
                buf, out_hbm.at[pl.ds(base, R)], sem).wait()

        fire_gather(0, buf_a, gsem_a)

        @pl.loop(0, n_slabs, step=2)
        def body(ja):
            jb = ja + 1
            wait_gather(buf_a, gsem_a)

            @pl.when(ja > 0)
            def _():
                wait_write(buf_b, wsem_b)

            fire_gather(jb, buf_b, gsem_b)
            hw_a = fire_write(ja, buf_a, wsem_a)
            wait_gather(buf_b, gsem_b)
            hw_a.wait()

            @pl.when(jb + 1 < n_slabs)
            def _():
                fire_gather(ja + 2, buf_a, gsem_a)

            fire_write(jb, buf_b, wsem_b)

        wait_write(buf_b, wsem_b)

    return emb


def kernel(input_ids, table):
    b, s = input_ids.shape
    v = table.shape[0]
    table2 = jnp.pad(table, ((0, 0), (0, D))).reshape(2 * v, D)
    return _build(b, s)(input_ids * 2, table2)
